# Initial kernel scaffold; baseline (speedup 1.0000x reference)
import jax
import jax.numpy as jnp
from jax.experimental import pallas as pl


def kernel(x, edge_index, W1, b1, W2, b2):
    raise NotImplementedError("write your pallas kernel here")


# speedup vs baseline: 9.3035x; 9.3035x over previous
import jax
import jax.numpy as jnp
from jax import lax
from jax.experimental import pallas as pl
from jax.experimental.pallas import tpu as pltpu
from jax.experimental.pallas import tpu_sc as plsc

N = 10000
NPAD = 10240
D = 128
E = 320000
EPAD = 327680
NC = 2
NS = 16
NW = NC * NS
C = 128
NCHUNK = EPAD // C
BPT = NCHUNK // NW
RPT = NPAD // NS
ZR = 64
BLK = 2000
NBLK = N // BLK


def _fill_f32(ref, nrows, val):
    v = jnp.full((16,), val, jnp.float32)

    def body(i, carry):
        for j in range(ref.shape[1] // 16):
            ref[i, pl.ds(j * 16, 16)] = v
        return carry

    lax.fori_loop(0, nrows, body, 0)


def _sc_deg_body(col_hbm, deg_hbm, col_v, dst_v, ones_v, z_v, deg_sm, sem):
    c = lax.axis_index("c")
    s = lax.axis_index("s")
    base_chunk = (c * NS + s) * BPT

    pltpu.sync_copy(col_hbm.at[pl.ds(base_chunk, BPT)], col_v)
    _fill_f32(ones_v, C, 1.0)
    _fill_f32(z_v, ZR, 0.0)
    for i in range(RPT // ZR):
        pltpu.sync_copy(z_v, deg_sm.at[pl.ds(s * RPT + i * ZR, ZR)])
    plsc.subcore_barrier()

    def batch(b, carry):
        for k in range(C // 16):
            dst_v[pl.ds(k * 16, 16)] = col_v[b, pl.ds(k * 16, 16)]
        pltpu.sync_copy(ones_v, deg_sm.at[dst_v], add=True)
        return carry

    lax.fori_loop(0, BPT, batch, 0)
    plsc.subcore_barrier()
    pltpu.sync_copy(deg_sm.at[pl.ds(s * RPT, RPT)],
                    deg_hbm.at[c, pl.ds(s * RPT, RPT)])


@jax.jit
def _sc_deg(col2d):
    mesh = plsc.VectorSubcoreMesh(core_axis_name="c", subcore_axis_name="s")
    return pl.kernel(
        _sc_deg_body,
        out_type=jax.ShapeDtypeStruct((NC, NPAD, D), jnp.float32),
        mesh=mesh,
        scratch_types=[
            pltpu.VMEM((BPT, C), jnp.int32),
            pltpu.VMEM((C,), jnp.int32),
            pltpu.VMEM((C, D), jnp.float32),
            pltpu.VMEM((ZR, D), jnp.float32),
            pltpu.VMEM_SHARED((NPAD, D), jnp.float32),
            pltpu.SemaphoreType.DMA,
        ],
    )(col2d)


def _sc_agg_body(pre_hbm, row_hbm, col_hbm, out_hbm,
                 row_v, col_v, idx_v, dst_v, rows_v, z_v, acc_sm, sem):
    c = lax.axis_index("c")
    s = lax.axis_index("s")
    base_chunk = (c * NS + s) * BPT

    pltpu.sync_copy(row_hbm.at[pl.ds(base_chunk, BPT)], row_v)
    pltpu.sync_copy(col_hbm.at[pl.ds(base_chunk, BPT)], col_v)
    _fill_f32(z_v, ZR, 0.0)
    for i in range(RPT // ZR):
        pltpu.sync_copy(z_v, acc_sm.at[pl.ds(s * RPT + i * ZR, ZR)])
    plsc.subcore_barrier()

    def batch(b, carry):
        for k in range(C // 16):
            idx_v[pl.ds(k * 16, 16)] = row_v[b, pl.ds(k * 16, 16)]
            dst_v[pl.ds(k * 16, 16)] = col_v[b, pl.ds(k * 16, 16)]
        pltpu.async_copy(pre_hbm.at[idx_v], rows_v, sem).wait()
        pltpu.sync_copy(rows_v, acc_sm.at[dst_v], add=True)
        return carry

    lax.fori_loop(0, BPT, batch, 0)
    plsc.subcore_barrier()
    pltpu.sync_copy(acc_sm.at[pl.ds(s * RPT, RPT)],
                    out_hbm.at[c, pl.ds(s * RPT, RPT)])


@jax.jit
def _sc_agg(pre, row2d, col2d):
    mesh = plsc.VectorSubcoreMesh(core_axis_name="c", subcore_axis_name="s")
    return pl.kernel(
        _sc_agg_body,
        out_type=jax.ShapeDtypeStruct((NC, NPAD, D), jnp.float32),
        mesh=mesh,
        scratch_types=[
            pltpu.VMEM((BPT, C), jnp.int32),
            pltpu.VMEM((BPT, C), jnp.int32),
            pltpu.VMEM((C,), jnp.int32),
            pltpu.VMEM((C,), jnp.int32),
            pltpu.VMEM((C, D), jnp.float32),
            pltpu.VMEM((ZR, D), jnp.float32),
            pltpu.VMEM_SHARED((NPAD, D), jnp.float32),
            pltpu.SemaphoreType.DMA,
        ],
    )(pre, row2d, col2d)


def _tc_prep_body(deg_ref, x_ref, w_ref, dinv_ref, pre_ref):
    dinvb = lax.rsqrt(deg_ref[0] + deg_ref[1] + 1.0)
    dinv_ref[...] = dinvb
    pre_ref[...] = dinvb * jnp.dot(x_ref[...], w_ref[...],
                                   preferred_element_type=jnp.float32)


@jax.jit
def _tc_prep(deg, x, W1):
    return pl.pallas_call(
        _tc_prep_body,
        grid=(NBLK,),
        in_specs=[
            pl.BlockSpec((NC, BLK, D), lambda i: (0, i, 0)),
            pl.BlockSpec((BLK, D), lambda i: (i, 0)),
            pl.BlockSpec((D, D), lambda i: (0, 0)),
        ],
        out_specs=[
            pl.BlockSpec((BLK, D), lambda i: (i, 0)),
            pl.BlockSpec((BLK, D), lambda i: (i, 0)),
        ],
        out_shape=[
            jax.ShapeDtypeStruct((N, D), jnp.float32),
            jax.ShapeDtypeStruct((N, D), jnp.float32),
        ],
    )(deg, x, W1)


def _tc_mid_body(acc_ref, pre_ref, dinv_ref, b_ref, w_ref, out_ref):
    agg = acc_ref[0] + acc_ref[1] + pre_ref[...]
    h = jnp.maximum(dinv_ref[...] * agg + b_ref[...], 0.0)
    out_ref[...] = dinv_ref[...] * jnp.dot(h, w_ref[...],
                                           preferred_element_type=jnp.float32)


@jax.jit
def _tc_mid(acc, pre, dinvb, b1, W2):
    return pl.pallas_call(
        _tc_mid_body,
        grid=(NBLK,),
        in_specs=[
            pl.BlockSpec((NC, BLK, D), lambda i: (0, i, 0)),
            pl.BlockSpec((BLK, D), lambda i: (i, 0)),
            pl.BlockSpec((BLK, D), lambda i: (i, 0)),
            pl.BlockSpec((1, D), lambda i: (0, 0)),
            pl.BlockSpec((D, D), lambda i: (0, 0)),
        ],
        out_specs=pl.BlockSpec((BLK, D), lambda i: (i, 0)),
        out_shape=jax.ShapeDtypeStruct((N, D), jnp.float32),
    )(acc, pre, dinvb, b1, W2)


def _tc_final_body(acc_ref, pre_ref, dinv_ref, b_ref, out_ref):
    agg = acc_ref[0] + acc_ref[1] + pre_ref[...]
    out_ref[...] = dinv_ref[...] * agg + b_ref[...]


@jax.jit
def _tc_final(acc, pre, dinvb, b2):
    return pl.pallas_call(
        _tc_final_body,
        grid=(NBLK,),
        in_specs=[
            pl.BlockSpec((NC, BLK, D), lambda i: (0, i, 0)),
            pl.BlockSpec((BLK, D), lambda i: (i, 0)),
            pl.BlockSpec((BLK, D), lambda i: (i, 0)),
            pl.BlockSpec((1, D), lambda i: (0, 0)),
        ],
        out_specs=pl.BlockSpec((BLK, D), lambda i: (i, 0)),
        out_shape=jax.ShapeDtypeStruct((N, D), jnp.float32),
    )(acc, pre, dinvb, b2)


def kernel(x, edge_index, W1, b1, W2, b2):
    npad_e = EPAD - E
    row = jnp.concatenate([edge_index[0], jnp.zeros((npad_e,), jnp.int32)])
    col = jnp.concatenate([edge_index[1], jnp.full((npad_e,), N, jnp.int32)])
    row2d = row.reshape(NCHUNK, C)
    col2d = col.reshape(NCHUNK, C)

    deg = _sc_deg(col2d)
    dinvb, pre1 = _tc_prep(deg[:, :N, :], x, W1)
    acc1 = _sc_agg(pre1, row2d, col2d)
    pre2 = _tc_mid(acc1[:, :N], pre1, dinvb, b1.reshape(1, D), W2)
    acc2 = _sc_agg(pre2, row2d, col2d)
    return _tc_final(acc2[:, :N], pre2, dinvb, b2.reshape(1, D))


# speedup vs baseline: 10.1058x; 1.0862x over previous
import jax
import jax.numpy as jnp
from jax import lax
from jax.experimental import pallas as pl
from jax.experimental.pallas import tpu as pltpu
from jax.experimental.pallas import tpu_sc as plsc

N = 10000
NPAD = 10240
D = 128
E = 320000
EPAD = 327680
NC = 2
NS = 16
NW = NC * NS
C = 128
NCHUNK = EPAD // C
BPT = NCHUNK // NW
RPT = NPAD // NS
ZR = 64
ZSM = 16
BLK = 2000
NBLK = N // BLK


def _fill_f32(ref, nrows, val):
    v = jnp.full((16,), val, jnp.float32)

    def body(i, carry):
        for j in range(ref.shape[1] // 16):
            ref[i, pl.ds(j * 16, 16)] = v
        return carry

    lax.fori_loop(0, nrows, body, 0)


def _sc_deg_body(col_hbm, deg_hbm, col_v, dst_v, ones_v, z_v, deg_sm, sem):
    c = lax.axis_index("c")
    s = lax.axis_index("s")
    base_chunk = (c * NS + s) * BPT

    pltpu.sync_copy(col_hbm.at[pl.ds(base_chunk, BPT)], col_v)
    _fill_f32(ones_v, C, 1.0)
    _fill_f32(z_v, ZR, 0.0)
    for i in range(RPT // ZR):
        pltpu.sync_copy(z_v, deg_sm.at[pl.ds(s * RPT + i * ZR, ZR)])
    plsc.subcore_barrier()

    def batch(b, carry):
        for k in range(C // 16):
            dst_v[pl.ds(k * 16, 16)] = col_v[b, pl.ds(k * 16, 16)]
        pltpu.sync_copy(ones_v, deg_sm.at[dst_v], add=True)
        return carry

    lax.fori_loop(0, BPT, batch, 0)
    plsc.subcore_barrier()
    pltpu.sync_copy(deg_sm.at[pl.ds(s * RPT, RPT)],
                    deg_hbm.at[c, pl.ds(s * RPT, RPT)])


@jax.jit
def _sc_deg(col2d):
    mesh = plsc.VectorSubcoreMesh(core_axis_name="c", subcore_axis_name="s")
    return pl.kernel(
        _sc_deg_body,
        out_type=jax.ShapeDtypeStruct((NC, NPAD, D), jnp.float32),
        mesh=mesh,
        scratch_types=[
            pltpu.VMEM((BPT, C), jnp.int32),
            pltpu.VMEM((C,), jnp.int32),
            pltpu.VMEM((C, D), jnp.float32),
            pltpu.VMEM((ZR, D), jnp.float32),
            pltpu.VMEM_SHARED((NPAD, D), jnp.float32),
            pltpu.SemaphoreType.DMA,
        ],
    )(col2d)


def _sc_agg_body(pre_hbm, row_hbm, col_hbm, out_hbm,
                 idx0_v, idx1_v, dst0_v, dst1_v,
                 rows0_v, rows1_v, z_v, acc_sm,
                 semg0, semg1, semi0, semi1):
    c = lax.axis_index("c")
    s = lax.axis_index("s")
    base_e = (c * NS + s) * BPT * C

    _fill_f32(z_v, ZSM, 0.0)
    for i in range(RPT // ZSM):
        pltpu.sync_copy(z_v, acc_sm.at[pl.ds(s * RPT + i * ZSM, ZSM)])
    plsc.subcore_barrier()

    idx_bufs = (idx0_v, idx1_v)
    dst_bufs = (dst0_v, dst1_v)
    rows_bufs = (rows0_v, rows1_v)
    semg = (semg0, semg1)
    semi = (semi0, semi1)

    def fire_idx(b, j):
        pltpu.async_copy(row_hbm.at[pl.ds(base_e + b * C, C)],
                         idx_bufs[j], semi[j])
        pltpu.async_copy(col_hbm.at[pl.ds(base_e + b * C, C)],
                         dst_bufs[j], semi[j])

    def wait_idx(j):
        pltpu.make_async_copy(row_hbm.at[pl.ds(base_e, C)],
                              idx_bufs[j], semi[j]).wait()
        pltpu.make_async_copy(col_hbm.at[pl.ds(base_e, C)],
                              dst_bufs[j], semi[j]).wait()

    def fire_g(j):
        pltpu.async_copy(pre_hbm.at[idx_bufs[j]], rows_bufs[j], semg[j])

    def wait_g(j):
        pltpu.make_async_copy(pre_hbm.at[idx_bufs[j]], rows_bufs[j],
                              semg[j]).wait()

    last = BPT - 1
    fire_idx(0, 0)
    wait_idx(0)
    fire_g(0)
    fire_idx(1, 1)
    wait_idx(1)
    fire_g(1)

    def pair(g, carry):
        b0 = g * 2
        wait_g(0)
        pltpu.sync_copy(rows_bufs[0], acc_sm.at[dst_bufs[0]], add=True)
        fire_idx(jnp.minimum(b0 + 2, last), 0)
        wait_idx(0)
        fire_g(0)
        wait_g(1)
        pltpu.sync_copy(rows_bufs[1], acc_sm.at[dst_bufs[1]], add=True)
        fire_idx(jnp.minimum(b0 + 3, last), 1)
        wait_idx(1)
        fire_g(1)
        return carry

    lax.fori_loop(0, BPT // 2, pair, 0)
    wait_g(0)
    wait_g(1)

    plsc.subcore_barrier()
    pltpu.sync_copy(acc_sm.at[pl.ds(s * RPT, RPT)],
                    out_hbm.at[c, pl.ds(s * RPT, RPT)])


@jax.jit
def _sc_agg(pre, row1d, col1d):
    mesh = plsc.VectorSubcoreMesh(core_axis_name="c", subcore_axis_name="s")
    return pl.kernel(
        _sc_agg_body,
        out_type=jax.ShapeDtypeStruct((NC, NPAD, D), jnp.float32),
        mesh=mesh,
        scratch_types=[
            pltpu.VMEM((C,), jnp.int32),
            pltpu.VMEM((C,), jnp.int32),
            pltpu.VMEM((C,), jnp.int32),
            pltpu.VMEM((C,), jnp.int32),
            pltpu.VMEM((C, D), jnp.float32),
            pltpu.VMEM((C, D), jnp.float32),
            pltpu.VMEM((ZSM, D), jnp.float32),
            pltpu.VMEM_SHARED((NPAD, D), jnp.float32),
            pltpu.SemaphoreType.DMA,
            pltpu.SemaphoreType.DMA,
            pltpu.SemaphoreType.DMA,
            pltpu.SemaphoreType.DMA,
        ],
    )(pre, row1d, col1d)


def _tc_prep_body(deg_ref, x_ref, w_ref, dinv_ref, pre_ref):
    dinvb = lax.rsqrt(deg_ref[0] + deg_ref[1] + 1.0)
    dinv_ref[...] = dinvb
    pre_ref[...] = dinvb * jnp.dot(x_ref[...], w_ref[...],
                                   preferred_element_type=jnp.float32)


@jax.jit
def _tc_prep(deg, x, W1):
    return pl.pallas_call(
        _tc_prep_body,
        grid=(NBLK,),
        in_specs=[
            pl.BlockSpec((NC, BLK, D), lambda i: (0, i, 0)),
            pl.BlockSpec((BLK, D), lambda i: (i, 0)),
            pl.BlockSpec((D, D), lambda i: (0, 0)),
        ],
        out_specs=[
            pl.BlockSpec((BLK, D), lambda i: (i, 0)),
            pl.BlockSpec((BLK, D), lambda i: (i, 0)),
        ],
        out_shape=[
            jax.ShapeDtypeStruct((N, D), jnp.float32),
            jax.ShapeDtypeStruct((N, D), jnp.float32),
        ],
    )(deg, x, W1)


def _tc_mid_body(acc_ref, pre_ref, dinv_ref, b_ref, w_ref, out_ref):
    agg = acc_ref[0] + acc_ref[1] + pre_ref[...]
    h = jnp.maximum(dinv_ref[...] * agg + b_ref[...], 0.0)
    out_ref[...] = dinv_ref[...] * jnp.dot(h, w_ref[...],
                                           preferred_element_type=jnp.float32)


@jax.jit
def _tc_mid(acc, pre, dinvb, b1, W2):
    return pl.pallas_call(
        _tc_mid_body,
        grid=(NBLK,),
        in_specs=[
            pl.BlockSpec((NC, BLK, D), lambda i: (0, i, 0)),
            pl.BlockSpec((BLK, D), lambda i: (i, 0)),
            pl.BlockSpec((BLK, D), lambda i: (i, 0)),
            pl.BlockSpec((1, D), lambda i: (0, 0)),
            pl.BlockSpec((D, D), lambda i: (0, 0)),
        ],
        out_specs=pl.BlockSpec((BLK, D), lambda i: (i, 0)),
        out_shape=jax.ShapeDtypeStruct((N, D), jnp.float32),
    )(acc, pre, dinvb, b1, W2)


def _tc_final_body(acc_ref, pre_ref, dinv_ref, b_ref, out_ref):
    agg = acc_ref[0] + acc_ref[1] + pre_ref[...]
    out_ref[...] = dinv_ref[...] * agg + b_ref[...]


@jax.jit
def _tc_final(acc, pre, dinvb, b2):
    return pl.pallas_call(
        _tc_final_body,
        grid=(NBLK,),
        in_specs=[
            pl.BlockSpec((NC, BLK, D), lambda i: (0, i, 0)),
            pl.BlockSpec((BLK, D), lambda i: (i, 0)),
            pl.BlockSpec((BLK, D), lambda i: (i, 0)),
            pl.BlockSpec((1, D), lambda i: (0, 0)),
        ],
        out_specs=pl.BlockSpec((BLK, D), lambda i: (i, 0)),
        out_shape=jax.ShapeDtypeStruct((N, D), jnp.float32),
    )(acc, pre, dinvb, b2)


def kernel(x, edge_index, W1, b1, W2, b2):
    npad_e = EPAD - E
    row = jnp.concatenate([edge_index[0], jnp.zeros((npad_e,), jnp.int32)])
    col = jnp.concatenate([edge_index[1], jnp.full((npad_e,), N, jnp.int32)])
    col2d = col.reshape(NCHUNK, C)

    deg = _sc_deg(col2d)
    dinvb, pre1 = _tc_prep(deg[:, :N, :], x, W1)
    acc1 = _sc_agg(pre1, row, col)
    pre2 = _tc_mid(acc1[:, :N], pre1, dinvb, b1.reshape(1, D), W2)
    acc2 = _sc_agg(pre2, row, col)
    return _tc_final(acc2[:, :N], pre2, dinvb, b2.reshape(1, D))
